# Initial kernel scaffold; baseline (speedup 1.0000x reference)
import jax
import jax.numpy as jnp
from jax.experimental import pallas as pl


def kernel(feat_idx, ifeat_idx, edge_index, cj, ci, review_feat, weight, prob_w):
    raise NotImplementedError("write your pallas kernel here")


# speedup vs baseline: 1.3717x; 1.3717x over previous
import functools
import jax
import jax.numpy as jnp
from jax import lax
from jax.experimental import pallas as pl
from jax.experimental.pallas import tpu as pltpu
from jax.experimental.pallas import tpu_sc as plsc

N = 50000
E = 800000
IN_FEATS = 50000
OUT_FEATS = 32
REVIEW_DIM = 64
NC = 2
NS = 16
L = 16

NGROUPS = 6
ROWS_PER_SUB = N // (NS)
ECHUNK = 128
E_PER_SC = E // NC
NCHUNKS_SC = E_PER_SC // ECHUNK
ACHUNK = 80
NCHUNKS_A = N // ACHUNK


def _pa_body(rf_ref, pw_ref, out_ref):
    x = rf_ref[...]
    w = pw_ref[...]
    s = jnp.sum(x * w, axis=1, keepdims=True)
    out_ref[...] = 1.0 / (1.0 + jnp.exp(-s))


def _pa_call(review_feat, prob_w):
    BE = 6400
    grid = E // BE
    return pl.pallas_call(
        _pa_body,
        grid=(grid,),
        in_specs=[
            pl.BlockSpec((BE, REVIEW_DIM), lambda i: (i, 0)),
            pl.BlockSpec((1, REVIEW_DIM), lambda i: (0, 0)),
        ],
        out_specs=pl.BlockSpec((BE, 1), lambda i: (i, 0)),
        out_shape=jax.ShapeDtypeStruct((E, 1), jnp.float32),
    )(review_feat, prob_w)


def _feat_builder(fidx0, fidx1, fidx2, cj, wh0, wh1):
    mesh = plsc.VectorSubcoreMesh(
        core_axis_name="c", subcore_axis_name="s", num_cores=NC, num_subcores=NS)

    @functools.partial(
        pl.kernel, mesh=mesh,
        compiler_params=pltpu.CompilerParams(use_tc_tiling_on_sc=False),
        out_type=[jax.ShapeDtypeStruct((N, L), jnp.float32) for _ in range(NGROUPS)],
        scratch_types=[
            pltpu.VMEM((ACHUNK,), jnp.int32),
            pltpu.VMEM((ACHUNK,), jnp.float32),
            pltpu.VMEM((ACHUNK, L), jnp.float32),
            pltpu.VMEM((ACHUNK, L), jnp.float32),
            pltpu.SemaphoreType.DMA,
            pltpu.SemaphoreType.DMA,
        ],
    )
    def k(f0_h, f1_h, f2_h, cj_h, wh0_h, wh1_h, o0, o1, o2, o3, o4, o5,
          idx_v, cj_v, rowsa_v, rowsb_v, sema, semb):
        cid = lax.axis_index("c")
        sid = lax.axis_index("s")
        wid = sid * NC + cid
        fidx = [f0_h, f1_h, f2_h]
        outs = [(o0, o1), (o2, o3), (o4, o5)]
        nw = NC * NS
        niter = (NCHUNKS_A - wid + nw - 1) // nw

        def chunk_body(i, _):
            base = (wid + i * nw) * ACHUNK
            pltpu.sync_copy(cj_h.at[pl.ds(base, ACHUNK)], cj_v)
            for j in range(3):
                pltpu.sync_copy(fidx[j].at[pl.ds(base, ACHUNK)], idx_v)
                cpa = pltpu.async_copy(wh0_h.at[idx_v], rowsa_v, sema)
                cpb = pltpu.async_copy(wh1_h.at[idx_v], rowsb_v, semb)
                cpa.wait()
                cpb.wait()

                for eb in range(ACHUNK // L):
                    cv = cj_v[pl.ds(eb * L, L)]
                    for e16 in range(L):
                        e = eb * L + e16
                        c = jnp.broadcast_to(cv[e16], (L,))
                        rowsa_v[e, :] = rowsa_v[e, :] * c
                        rowsb_v[e, :] = rowsb_v[e, :] * c
                pltpu.sync_copy(rowsa_v, outs[j][0].at[pl.ds(base, ACHUNK)])
                pltpu.sync_copy(rowsb_v, outs[j][1].at[pl.ds(base, ACHUNK)])
            return 0

        lax.fori_loop(0, niter, chunk_body, 0)

    return k(fidx0, fidx1, fidx2, cj, wh0, wh1)


def _message_pass(src, dst, pa, feats):
    mesh = plsc.VectorSubcoreMesh(
        core_axis_name="c", subcore_axis_name="s", num_cores=NC, num_subcores=NS)

    @functools.partial(
        pl.kernel, mesh=mesh,
        compiler_params=pltpu.CompilerParams(use_tc_tiling_on_sc=False),
        out_type=jax.ShapeDtypeStruct((NC, N, NGROUPS * L), jnp.float32),
        scratch_types=[
            pltpu.VMEM((ROWS_PER_SUB, L), jnp.float32),
            pltpu.VMEM((ECHUNK,), jnp.int32),
            pltpu.VMEM((ECHUNK,), jnp.int32),
            pltpu.VMEM((ECHUNK,), jnp.float32),
            pltpu.VMEM((ECHUNK, L), jnp.float32),
            pltpu.VMEM_SHARED((N, L), jnp.float32),
            pltpu.SemaphoreType.DMA,
        ],
    )
    def k(src_h, dst_h, pa_h, f0_h, f1_h, f2_h, f3_h, f4_h, f5_h, out_h,
          zbuf_v, sidx_v, didx_v, pa_v, rows_v, h_sh, sem):
        cid = lax.axis_index("c")
        sid = lax.axis_index("s")
        fgs = [f0_h, f1_h, f2_h, f3_h, f4_h, f5_h]
        def zfill(i, _):
            zbuf_v[i, :] = jnp.zeros((L,), jnp.float32)
            return 0
        lax.fori_loop(0, ROWS_PER_SUB, zfill, 0)

        nchunks = (NCHUNKS_SC - sid + NS - 1) // NS
        ebase0 = cid * E_PER_SC
        row0 = sid * ROWS_PER_SUB

        for g in range(NGROUPS):
            pltpu.sync_copy(zbuf_v, h_sh.at[pl.ds(row0, ROWS_PER_SUB)])
            plsc.subcore_barrier()

            def chunk_body(i, _):
                base = ebase0 + (sid + i * NS) * ECHUNK
                pltpu.sync_copy(src_h.at[pl.ds(base, ECHUNK)], sidx_v)
                pltpu.sync_copy(dst_h.at[pl.ds(base, ECHUNK)], didx_v)
                pltpu.sync_copy(pa_h.at[pl.ds(base, ECHUNK)], pa_v)
                pltpu.async_copy(fgs[g].at[sidx_v], rows_v, sem).wait()

                for eb in range(ECHUNK // L):
                    pv = pa_v[pl.ds(eb * L, L)]
                    for e16 in range(L):
                        e = eb * L + e16
                        rows_v[e, :] = rows_v[e, :] * jnp.broadcast_to(pv[e16], (L,))
                pltpu.sync_copy(rows_v, h_sh.at[didx_v], add=True)
                return 0

            lax.fori_loop(0, nchunks, chunk_body, 0)
            plsc.subcore_barrier()
            pltpu.sync_copy(
                h_sh.at[pl.ds(row0, ROWS_PER_SUB)],
                out_h.at[cid, pl.ds(row0, ROWS_PER_SUB), pl.ds(g * L, L)])
        plsc.subcore_barrier()

    return k(src, dst, pa, *feats)


def _combine_body(part_ref, ci_ref, out_ref):
    x = part_ref[...]
    s = x[0] + x[1]
    out_ref[...] = s * ci_ref[...]


def _combine(part, ci):
    BN = 1000
    grid = N // BN
    return pl.pallas_call(
        _combine_body,
        grid=(grid,),
        in_specs=[
            pl.BlockSpec((NC, BN, NGROUPS * L), lambda i: (0, i, 0)),
            pl.BlockSpec((BN, 1), lambda i: (i, 0)),
        ],
        out_specs=pl.BlockSpec((BN, NGROUPS * L), lambda i: (i, 0)),
        out_shape=jax.ShapeDtypeStruct((N, NGROUPS * L), jnp.float32),
    )(part, ci)


def kernel(feat_idx, ifeat_idx, edge_index, cj, ci, review_feat, weight, prob_w):
    del ifeat_idx
    fidx0 = feat_idx[:, 0].astype(jnp.int32)
    fidx1 = feat_idx[:, 1].astype(jnp.int32)
    fidx2 = feat_idx[:, 2].astype(jnp.int32)
    src = edge_index[0].astype(jnp.int32)
    dst = edge_index[1].astype(jnp.int32)
    cjf = cj.reshape(N)
    wh0 = weight[:, :L]
    wh1 = weight[:, L:]

    pa = _pa_call(review_feat, prob_w)
    feats = _feat_builder(fidx0, fidx1, fidx2, cjf, wh0, wh1)
    part = _message_pass(src, dst, pa.reshape(E), feats)
    return _combine(part, ci)


# speedup vs baseline: 2.3694x; 1.7274x over previous
import functools
import jax
import jax.numpy as jnp
from jax import lax
from jax.experimental import pallas as pl
from jax.experimental.pallas import tpu as pltpu
from jax.experimental.pallas import tpu_sc as plsc

N = 50000
E = 800000
IN_FEATS = 50000
OUT_FEATS = 32
REVIEW_DIM = 64
NC = 2
NS = 16
L = 16

NGROUPS = 6
ROWS_PER_SUB = N // (NS)
ECHUNK = 128
CH_PER_SUB = 200
NCH_TOT = NC * NS * CH_PER_SUB
E_PAD = NCH_TOT * ECHUNK
NBUF = 4
ACHUNK = 80
NCHUNKS_A = N // ACHUNK


def _pa_body(rf_ref, pw_ref, out_ref):
    x = rf_ref[...]
    w = pw_ref[...]
    s = jnp.sum(x * w, axis=1, keepdims=True)
    out_ref[...] = 1.0 / (1.0 + jnp.exp(-s))


def _pa_call(review_feat, prob_w):
    BE = 6400
    grid = E // BE
    return pl.pallas_call(
        _pa_body,
        grid=(grid,),
        in_specs=[
            pl.BlockSpec((BE, REVIEW_DIM), lambda i: (i, 0)),
            pl.BlockSpec((1, REVIEW_DIM), lambda i: (0, 0)),
        ],
        out_specs=pl.BlockSpec((BE, 1), lambda i: (i, 0)),
        out_shape=jax.ShapeDtypeStruct((E, 1), jnp.float32),
    )(review_feat, prob_w)


def _feat_builder(fidx0, fidx1, fidx2, cj, wh0, wh1):
    mesh = plsc.VectorSubcoreMesh(
        core_axis_name="c", subcore_axis_name="s", num_cores=NC, num_subcores=NS)

    @functools.partial(
        pl.kernel, mesh=mesh,
        compiler_params=pltpu.CompilerParams(use_tc_tiling_on_sc=False, needs_layout_passes=False),
        out_type=jax.ShapeDtypeStruct((NGROUPS, N, L), jnp.float32),
        scratch_types=[
            pltpu.VMEM((ACHUNK,), jnp.int32),
            pltpu.VMEM((ACHUNK,), jnp.float32),
            pltpu.VMEM((ACHUNK, L), jnp.float32),
            pltpu.VMEM((ACHUNK, L), jnp.float32),
            pltpu.SemaphoreType.DMA,
            pltpu.SemaphoreType.DMA,
        ],
    )
    def k(f0_h, f1_h, f2_h, cj_h, wh0_h, wh1_h, o6,
          idx_v, cj_v, rowsa_v, rowsb_v, sema, semb):
        cid = lax.axis_index("c")
        sid = lax.axis_index("s")
        wid = sid * NC + cid
        fidx = [f0_h, f1_h, f2_h]
        nw = NC * NS
        niter = (NCHUNKS_A - wid + nw - 1) // nw

        def chunk_body(i, _):
            base = (wid + i * nw) * ACHUNK
            pltpu.sync_copy(cj_h.at[pl.ds(base, ACHUNK)], cj_v)
            for j in range(3):
                pltpu.sync_copy(fidx[j].at[pl.ds(base, ACHUNK)], idx_v)
                cpa = pltpu.async_copy(wh0_h.at[idx_v], rowsa_v, sema)
                cpb = pltpu.async_copy(wh1_h.at[idx_v], rowsb_v, semb)
                cpa.wait()
                cpb.wait()

                for eb in range(ACHUNK // L):
                    cv = cj_v[pl.ds(eb * L, L)]
                    for e16 in range(L):
                        e = eb * L + e16
                        c = jnp.broadcast_to(cv[e16], (L,))
                        rowsa_v[e, :] = rowsa_v[e, :] * c
                        rowsb_v[e, :] = rowsb_v[e, :] * c
                pltpu.sync_copy(rowsa_v, o6.at[2 * j, pl.ds(base, ACHUNK)])
                pltpu.sync_copy(rowsb_v, o6.at[2 * j + 1, pl.ds(base, ACHUNK)])
            return 0

        lax.fori_loop(0, niter, chunk_body, 0)

    return k(fidx0, fidx1, fidx2, cj, wh0, wh1)


def _message_pass(meta, f6):
    mesh = plsc.VectorSubcoreMesh(
        core_axis_name="c", subcore_axis_name="s", num_cores=NC, num_subcores=NS)

    @functools.partial(
        pl.kernel, mesh=mesh,
        compiler_params=pltpu.CompilerParams(use_tc_tiling_on_sc=False,
                                             needs_layout_passes=False),
        out_type=jax.ShapeDtypeStruct((NC, NGROUPS, N, L), jnp.float32),
        scratch_types=[
            pltpu.VMEM((ROWS_PER_SUB, L), jnp.float32),
            pltpu.VMEM((NBUF, 3, ECHUNK), jnp.int32),
            pltpu.VMEM((NBUF, ECHUNK, L), jnp.float32),
            pltpu.VMEM((NBUF, ECHUNK), jnp.int32),
            pltpu.VMEM_SHARED((N, L), jnp.float32),
            [pltpu.SemaphoreType.DMA for _ in range(NBUF)],
            [pltpu.SemaphoreType.DMA for _ in range(NBUF)],
            [pltpu.SemaphoreType.DMA for _ in range(NBUF)],
        ],
    )
    def k(meta_h, f6_h, out_h,
          zbuf_v, meta_v, rows_v, didx_v, h_sh, msems, gsems, ssems):
        cid = lax.axis_index("c")
        sid = lax.axis_index("s")

        def zfill(i, _):
            zbuf_v[i, :] = jnp.zeros((L,), jnp.float32)
            return 0
        lax.fori_loop(0, ROWS_PER_SUB, zfill, 0)

        k0 = (cid * NS + sid) * CH_PER_SUB
        kmax = NCH_TOT - 1
        row0 = sid * ROWS_PER_SUB

        def fire_meta(i, b):
            kk = jnp.minimum(k0 + i, kmax)
            pltpu.async_copy(meta_h.at[kk], meta_v.at[b], msems[b])

        def wait_meta(b):
            pltpu.make_async_copy(meta_h.at[0], meta_v.at[b], msems[b]).wait()

        def fire_gather(g, b):
            pltpu.async_copy(
                f6_h.at[g].at[meta_v.at[b, 0]], rows_v.at[b], gsems[b])

        def wait_gather(g, b):
            pltpu.make_async_copy(
                f6_h.at[g].at[meta_v.at[b, 0]], rows_v.at[b], gsems[b]).wait()

        def fire_scatter(b):
            pltpu.async_copy(rows_v.at[b], h_sh.at[didx_v.at[b]], ssems[b],
                             add=True)

        def wait_scatter(b):
            pltpu.make_async_copy(
                rows_v.at[b], h_sh.at[didx_v.at[b]], ssems[b]).wait()

        def scale(b):
            for eb in range(ECHUNK // L):
                didx_v[b, pl.ds(eb * L, L)] = meta_v[b, 1, pl.ds(eb * L, L)]
                pv = plsc.bitcast(meta_v[b, 2, pl.ds(eb * L, L)], jnp.float32)
                for e16 in range(L):
                    e = eb * L + e16
                    rows_v[b, e, :] = rows_v[b, e, :] * jnp.broadcast_to(
                        pv[e16], (L,))

        def group_body(g, _):
            pltpu.sync_copy(zbuf_v, h_sh.at[pl.ds(row0, ROWS_PER_SUB)])
            plsc.subcore_barrier()

            zv = jnp.zeros((L,), jnp.float32)
            for e in range(ECHUNK):
                rows_v[NBUF - 1, e, :] = zv
            for eb in range(ECHUNK // L):
                didx_v[NBUF - 1, pl.ds(eb * L, L)] = jnp.zeros((L,), jnp.int32)
            fire_scatter(NBUF - 1)

            for b in range(NBUF - 1):
                fire_meta(b, b)
            wait_meta(0)
            fire_gather(g, 0)
            wait_meta(1)
            fire_gather(g, 1)

            def block_loop(i4, _):
                for j in range(NBUF):
                    i = i4 * NBUF + j
                    wait_gather(g, j)
                    scale(j)
                    fire_scatter(j)
                    wait_scatter((j + 3) % NBUF)
                    wait_meta((j + 2) % NBUF)
                    fire_gather(g, (j + 2) % NBUF)
                    fire_meta(i + 3, (j + 3) % NBUF)
                return 0

            lax.fori_loop(0, CH_PER_SUB // NBUF, block_loop, 0)

            wait_scatter((CH_PER_SUB - 1) % NBUF)
            wait_gather(g, CH_PER_SUB % NBUF)
            wait_gather(g, (CH_PER_SUB + 1) % NBUF)
            wait_meta((CH_PER_SUB + 2) % NBUF)

            plsc.subcore_barrier()
            pltpu.sync_copy(
                h_sh.at[pl.ds(row0, ROWS_PER_SUB)],
                out_h.at[cid, g, pl.ds(row0, ROWS_PER_SUB)])
            return 0

        lax.fori_loop(0, NGROUPS, group_body, 0)
        plsc.subcore_barrier()

    return k(meta, f6)


def _combine_body(part_ref, ci_ref, out_ref):
    x = part_ref[...]
    s = x[0] + x[1]
    out_ref[...] = s * ci_ref[...][None, :, :]


def _combine(part, ci):
    BN = 1000
    grid = N // BN
    return pl.pallas_call(
        _combine_body,
        grid=(grid,),
        in_specs=[
            pl.BlockSpec((NC, NGROUPS, BN, L), lambda i: (0, 0, i, 0)),
            pl.BlockSpec((BN, 1), lambda i: (i, 0)),
        ],
        out_specs=pl.BlockSpec((NGROUPS, BN, L), lambda i: (0, i, 0)),
        out_shape=jax.ShapeDtypeStruct((NGROUPS, N, L), jnp.float32),
    )(part, ci)


def kernel(feat_idx, ifeat_idx, edge_index, cj, ci, review_feat, weight, prob_w):
    del ifeat_idx
    fidx0 = feat_idx[:, 0].astype(jnp.int32)
    fidx1 = feat_idx[:, 1].astype(jnp.int32)
    fidx2 = feat_idx[:, 2].astype(jnp.int32)
    src = edge_index[0].astype(jnp.int32)
    dst = edge_index[1].astype(jnp.int32)
    cjf = cj.reshape(N)
    wh0 = weight[:, :L]
    wh1 = weight[:, L:]

    pa = _pa_call(review_feat, prob_w).reshape(E)
    pad = E_PAD - E
    srcp = jnp.concatenate([src, jnp.zeros((pad,), jnp.int32)])
    dstp = jnp.concatenate([dst, jnp.zeros((pad,), jnp.int32)])
    pap = jnp.concatenate([pa, jnp.zeros((pad,), jnp.float32)])
    meta = jnp.stack([
        srcp.reshape(NCH_TOT, ECHUNK),
        dstp.reshape(NCH_TOT, ECHUNK),
        lax.bitcast_convert_type(pap, jnp.int32).reshape(NCH_TOT, ECHUNK),
    ], axis=1)

    f6 = _feat_builder(fidx0, fidx1, fidx2, cjf, wh0, wh1)
    part = _message_pass(meta, f6)
    out6 = _combine(part, ci)
    return out6.transpose(1, 0, 2).reshape(N, NGROUPS * L)


# speedup vs baseline: 2.6451x; 1.1163x over previous
import functools
import jax
import jax.numpy as jnp
from jax import lax
from jax.experimental import pallas as pl
from jax.experimental.pallas import tpu as pltpu
from jax.experimental.pallas import tpu_sc as plsc

N = 50000
E = 800000
IN_FEATS = 50000
OUT_FEATS = 32
REVIEW_DIM = 64
NC = 2
NS = 16
L = 16

NGROUPS = 6
ROWS_PER_SUB = N // (NS)
ECHUNK = 128
CH_PER_SUB = 200
NCH_TOT = NC * NS * CH_PER_SUB
E_PAD = NCH_TOT * ECHUNK
NBUF = 4
ACHUNK = 80
NCHUNKS_A = N // ACHUNK


CPB = 50
BE_META = CPB * ECHUNK
NCH_REAL = E // ECHUNK


def _meta_body(ei_ref, rf_ref, pw_ref, out_ref):
    x = rf_ref[...]
    w = pw_ref[...]
    sv = jnp.sum(x * w, axis=1)
    pa = 1.0 / (1.0 + jnp.exp(-sv))
    bits = lax.bitcast_convert_type(pa, jnp.int32)
    src = ei_ref[0]
    dst = ei_ref[1]
    out_ref[...] = jnp.stack([
        src.reshape(CPB, ECHUNK),
        dst.reshape(CPB, ECHUNK),
        bits.reshape(CPB, ECHUNK),
    ], axis=1)


def _meta_call(edge_index, review_feat, prob_w):
    grid = E // BE_META
    return pl.pallas_call(
        _meta_body,
        grid=(grid,),
        in_specs=[
            pl.BlockSpec((2, BE_META), lambda i: (0, i)),
            pl.BlockSpec((BE_META, REVIEW_DIM), lambda i: (i, 0)),
            pl.BlockSpec((1, REVIEW_DIM), lambda i: (0, 0)),
        ],
        out_specs=pl.BlockSpec((CPB, 3, ECHUNK), lambda i: (i, 0, 0)),
        out_shape=jax.ShapeDtypeStruct((NCH_REAL, 3, ECHUNK), jnp.int32),
    )(edge_index, review_feat, prob_w)


def _feat_builder(fidx0, fidx1, fidx2, cj, wh0, wh1):
    mesh = plsc.VectorSubcoreMesh(
        core_axis_name="c", subcore_axis_name="s", num_cores=NC, num_subcores=NS)

    @functools.partial(
        pl.kernel, mesh=mesh,
        compiler_params=pltpu.CompilerParams(use_tc_tiling_on_sc=False, needs_layout_passes=False),
        out_type=jax.ShapeDtypeStruct((NGROUPS, N, L), jnp.float32),
        scratch_types=[
            pltpu.VMEM((ACHUNK,), jnp.int32),
            pltpu.VMEM((ACHUNK,), jnp.float32),
            pltpu.VMEM((ACHUNK, L), jnp.float32),
            pltpu.VMEM((ACHUNK, L), jnp.float32),
            pltpu.SemaphoreType.DMA,
            pltpu.SemaphoreType.DMA,
        ],
    )
    def k(f0_h, f1_h, f2_h, cj_h, wh0_h, wh1_h, o6,
          idx_v, cj_v, rowsa_v, rowsb_v, sema, semb):
        cid = lax.axis_index("c")
        sid = lax.axis_index("s")
        wid = sid * NC + cid
        fidx = [f0_h, f1_h, f2_h]
        nw = NC * NS
        niter = (NCHUNKS_A - wid + nw - 1) // nw

        def chunk_body(i, _):
            base = (wid + i * nw) * ACHUNK
            pltpu.sync_copy(cj_h.at[pl.ds(base, ACHUNK)], cj_v)
            for j in range(3):
                pltpu.sync_copy(fidx[j].at[pl.ds(base, ACHUNK)], idx_v)
                cpa = pltpu.async_copy(wh0_h.at[idx_v], rowsa_v, sema)
                cpb = pltpu.async_copy(wh1_h.at[idx_v], rowsb_v, semb)
                cpa.wait()
                cpb.wait()

                for eb in range(ACHUNK // L):
                    cv = cj_v[pl.ds(eb * L, L)]
                    for e16 in range(L):
                        e = eb * L + e16
                        c = jnp.broadcast_to(cv[e16], (L,))
                        rowsa_v[e, :] = rowsa_v[e, :] * c
                        rowsb_v[e, :] = rowsb_v[e, :] * c
                pltpu.sync_copy(rowsa_v, o6.at[2 * j, pl.ds(base, ACHUNK)])
                pltpu.sync_copy(rowsb_v, o6.at[2 * j + 1, pl.ds(base, ACHUNK)])
            return 0

        lax.fori_loop(0, niter, chunk_body, 0)

    return k(fidx0, fidx1, fidx2, cj, wh0, wh1)


def _message_pass(meta, f6):
    mesh = plsc.VectorSubcoreMesh(
        core_axis_name="c", subcore_axis_name="s", num_cores=NC, num_subcores=NS)

    @functools.partial(
        pl.kernel, mesh=mesh,
        compiler_params=pltpu.CompilerParams(use_tc_tiling_on_sc=False,
                                             needs_layout_passes=False),
        out_type=jax.ShapeDtypeStruct((NC, NGROUPS, N, L), jnp.float32),
        scratch_types=[
            pltpu.VMEM((ROWS_PER_SUB, L), jnp.float32),
            pltpu.VMEM((NBUF, 3, ECHUNK), jnp.int32),
            pltpu.VMEM((NBUF, ECHUNK, L), jnp.float32),
            pltpu.VMEM((NBUF, ECHUNK), jnp.int32),
            pltpu.VMEM_SHARED((N, L), jnp.float32),
            [pltpu.SemaphoreType.DMA for _ in range(NBUF)],
            [pltpu.SemaphoreType.DMA for _ in range(NBUF)],
            [pltpu.SemaphoreType.DMA for _ in range(NBUF)],
        ],
    )
    def k(meta_h, f6_h, out_h,
          zbuf_v, meta_v, rows_v, didx_v, h_sh, msems, gsems, ssems):
        cid = lax.axis_index("c")
        sid = lax.axis_index("s")

        def zfill(i, _):
            zbuf_v[i, :] = jnp.zeros((L,), jnp.float32)
            return 0
        lax.fori_loop(0, ROWS_PER_SUB, zfill, 0)

        k0 = (cid * NS + sid) * CH_PER_SUB
        kmax = NCH_TOT - 1
        row0 = sid * ROWS_PER_SUB

        def fire_meta(i, b):
            kk = jnp.minimum(k0 + i, kmax)
            pltpu.async_copy(meta_h.at[kk], meta_v.at[b], msems[b])

        def wait_meta(b):
            pltpu.make_async_copy(meta_h.at[0], meta_v.at[b], msems[b]).wait()

        def fire_gather(g, b):
            pltpu.async_copy(
                f6_h.at[g].at[meta_v.at[b, 0]], rows_v.at[b], gsems[b])

        def wait_gather(g, b):
            pltpu.make_async_copy(
                f6_h.at[g].at[meta_v.at[b, 0]], rows_v.at[b], gsems[b]).wait()

        def fire_scatter(b):
            pltpu.async_copy(rows_v.at[b], h_sh.at[didx_v.at[b]], ssems[b],
                             add=True)

        def wait_scatter(b):
            pltpu.make_async_copy(
                rows_v.at[b], h_sh.at[didx_v.at[b]], ssems[b]).wait()

        def scale(b):
            for eb in range(ECHUNK // L):
                didx_v[b, pl.ds(eb * L, L)] = meta_v[b, 1, pl.ds(eb * L, L)]
                pv = plsc.bitcast(meta_v[b, 2, pl.ds(eb * L, L)], jnp.float32)
                for e16 in range(L):
                    e = eb * L + e16
                    rows_v[b, e, :] = rows_v[b, e, :] * jnp.broadcast_to(
                        pv[e16], (L,))

        def group_body(g, _):
            pltpu.sync_copy(zbuf_v, h_sh.at[pl.ds(row0, ROWS_PER_SUB)])
            plsc.subcore_barrier()

            zv = jnp.zeros((L,), jnp.float32)
            for e in range(ECHUNK):
                rows_v[NBUF - 1, e, :] = zv
            for eb in range(ECHUNK // L):
                didx_v[NBUF - 1, pl.ds(eb * L, L)] = jnp.zeros((L,), jnp.int32)
            fire_scatter(NBUF - 1)

            for b in range(NBUF - 1):
                fire_meta(b, b)
            wait_meta(0)
            fire_gather(g, 0)
            wait_meta(1)
            fire_gather(g, 1)

            def block_loop(i4, _):
                for j in range(NBUF):
                    i = i4 * NBUF + j
                    wait_gather(g, j)
                    scale(j)
                    fire_scatter(j)
                    wait_scatter((j + 3) % NBUF)
                    wait_meta((j + 2) % NBUF)
                    fire_gather(g, (j + 2) % NBUF)
                    fire_meta(i + 3, (j + 3) % NBUF)
                return 0

            lax.fori_loop(0, CH_PER_SUB // NBUF, block_loop, 0)

            wait_scatter((CH_PER_SUB - 1) % NBUF)
            wait_gather(g, CH_PER_SUB % NBUF)
            wait_gather(g, (CH_PER_SUB + 1) % NBUF)
            wait_meta((CH_PER_SUB + 2) % NBUF)

            plsc.subcore_barrier()
            pltpu.sync_copy(
                h_sh.at[pl.ds(row0, ROWS_PER_SUB)],
                out_h.at[cid, g, pl.ds(row0, ROWS_PER_SUB)])
            return 0

        lax.fori_loop(0, NGROUPS, group_body, 0)
        plsc.subcore_barrier()

    return k(meta, f6)


def _combine_body(part_ref, ci_ref, out_ref):
    x = part_ref[...]
    s = x[0] + x[1]
    out_ref[...] = s * ci_ref[...][None, :, :]


def _combine(part, ci):
    BN = 1000
    grid = N // BN
    return pl.pallas_call(
        _combine_body,
        grid=(grid,),
        in_specs=[
            pl.BlockSpec((NC, NGROUPS, BN, L), lambda i: (0, 0, i, 0)),
            pl.BlockSpec((BN, 1), lambda i: (i, 0)),
        ],
        out_specs=pl.BlockSpec((NGROUPS, BN, L), lambda i: (0, i, 0)),
        out_shape=jax.ShapeDtypeStruct((NGROUPS, N, L), jnp.float32),
    )(part, ci)


def kernel(feat_idx, ifeat_idx, edge_index, cj, ci, review_feat, weight, prob_w):
    del ifeat_idx
    fidx0 = feat_idx[:, 0].astype(jnp.int32)
    fidx1 = feat_idx[:, 1].astype(jnp.int32)
    fidx2 = feat_idx[:, 2].astype(jnp.int32)
    cjf = cj.reshape(N)
    wh0 = weight[:, :L]
    wh1 = weight[:, L:]

    meta_real = _meta_call(edge_index.astype(jnp.int32), review_feat, prob_w)
    meta = jnp.concatenate(
        [meta_real,
         jnp.zeros((NCH_TOT - NCH_REAL, 3, ECHUNK), jnp.int32)], axis=0)

    f6 = _feat_builder(fidx0, fidx1, fidx2, cjf, wh0, wh1)
    part = _message_pass(meta, f6)
    out6 = _combine(part, ci)
    return out6.transpose(1, 0, 2).reshape(N, NGROUPS * L)
